# baseline (device time: 56905 ns/iter reference)
import jax
import jax.numpy as jnp
from jax import lax
from jax.experimental import pallas as pl
from jax.experimental.pallas import tpu as pltpu

_sem_signal = getattr(pl, "semaphore_signal", None) or pltpu.semaphore_signal
_sem_wait = getattr(pl, "semaphore_wait", None) or pltpu.semaphore_wait
_DeviceIdType = getattr(pl, "DeviceIdType", None) or pltpu.DeviceIdType
_CompilerParams = getattr(pltpu, "CompilerParams", None) or getattr(
    pltpu, "TPUCompilerParams"
)

M = 1024
D = 1024
EPS = 1e-6


def kernel(partial, resid, gamma):
    p = partial.reshape(M, D)
    g = gamma.reshape(1, D)

    def body(p_ref, r_ref, g_ref, out_ref, comm_ref, send_sem, recv_sem):
        my_x = lax.axis_index("x")
        my_y = lax.axis_index("y")
        my_z = lax.axis_index("z")
        nbr = (1 - my_x, my_y, my_z)

        barrier_sem = pltpu.get_barrier_semaphore()
        _sem_signal(
            barrier_sem, inc=1, device_id=nbr, device_id_type=_DeviceIdType.MESH
        )
        _sem_wait(barrier_sem, 1)

        rdma = pltpu.make_async_remote_copy(
            src_ref=p_ref,
            dst_ref=comm_ref,
            send_sem=send_sem,
            recv_sem=recv_sem,
            device_id=nbr,
            device_id_type=_DeviceIdType.MESH,
        )
        rdma.start()
        rdma.wait()

        y = p_ref[...] + comm_ref[...] + r_ref[...]
        ms = jnp.mean(y * y, axis=-1, keepdims=True)
        out_ref[...] = y * lax.rsqrt(ms + EPS) * g_ref[...]

    return pl.pallas_call(
        body,
        out_shape=jax.ShapeDtypeStruct((M, D), jnp.float32),
        in_specs=[
            pl.BlockSpec(memory_space=pltpu.VMEM),
            pl.BlockSpec(memory_space=pltpu.VMEM),
            pl.BlockSpec(memory_space=pltpu.VMEM),
        ],
        out_specs=pl.BlockSpec(memory_space=pltpu.VMEM),
        scratch_shapes=[
            pltpu.VMEM((M, D), jnp.float32),
            pltpu.SemaphoreType.DMA,
            pltpu.SemaphoreType.DMA,
        ],
        compiler_params=_CompilerParams(collective_id=0),
    )(p, resid, g)


# device time: 34507 ns/iter; 1.6491x vs baseline; 1.6491x over previous
import jax
import jax.numpy as jnp
from jax import lax
from jax.experimental import pallas as pl
from jax.experimental.pallas import tpu as pltpu

_sem_signal = getattr(pl, "semaphore_signal", None) or pltpu.semaphore_signal
_sem_wait = getattr(pl, "semaphore_wait", None) or pltpu.semaphore_wait
_DeviceIdType = getattr(pl, "DeviceIdType", None) or pltpu.DeviceIdType
_CompilerParams = getattr(pltpu, "CompilerParams", None) or getattr(
    pltpu, "TPUCompilerParams"
)

M = 1024
D = 1024
EPS = 1e-6
NB = 4
BM = M // NB
C = 4
CM = BM // C


def kernel(partial, resid, gamma):
    p = partial.reshape(M, D)
    g = gamma.reshape(1, D)

    def body(
        p_ref, r_ref, g_ref, out_ref, comm_x,
        sx_s, sx_r, yd_s, yd_r, zd_s, zd_r, yf_s, yf_r, zf_s, zf_r,
    ):
        my_x = lax.axis_index("x")
        my_y = lax.axis_index("y")
        my_z = lax.axis_index("z")
        xn = (1 - my_x, my_y, my_z)
        yn = (my_x, 1 - my_y, my_z)
        zn = (my_x, my_y, 1 - my_z)

        b = 2 * my_y + my_z
        b_y = 2 * (1 - my_y) + my_z
        b_z = 2 * my_y + (1 - my_z)

        def rows(blk, c):
            return pl.ds(blk * BM + c * CM, CM)

        barrier_sem = pltpu.get_barrier_semaphore()
        for nbr in (xn, yn, zn):
            _sem_signal(
                barrier_sem, inc=1, device_id=nbr,
                device_id_type=_DeviceIdType.MESH,
            )
        _sem_wait(barrier_sem, 3)

        x_rdma = []
        for c in range(C):
            r = pltpu.make_async_remote_copy(
                src_ref=p_ref.at[rows(b, c), :],
                dst_ref=comm_x.at[pl.ds(c * CM, CM), :],
                send_sem=sx_s.at[c],
                recv_sem=sx_r.at[c],
                device_id=xn,
                device_id_type=_DeviceIdType.MESH,
            )
            r.start()
            x_rdma.append(r)

        def direct(blk, c, sems_s, sems_r, nbr):
            return pltpu.make_async_remote_copy(
                src_ref=out_ref.at[rows(blk, c), :],
                dst_ref=out_ref.at[rows(blk, c), :],
                send_sem=sems_s.at[c],
                recv_sem=sems_r.at[c],
                device_id=nbr,
                device_id_type=_DeviceIdType.MESH,
            )

        yd, zd = [], []
        for c in range(C):
            x_rdma[c].wait_recv()
            y = p_ref[rows(b, c), :] + comm_x[pl.ds(c * CM, CM), :] \
                + r_ref[rows(b, c), :]
            ms = jnp.mean(y * y, axis=-1, keepdims=True)
            out_ref[rows(b, c), :] = y * lax.rsqrt(ms + EPS) * g_ref[...]
            ryd = direct(b, c, yd_s, yd_r, yn)
            rzd = direct(b, c, zd_s, zd_r, zn)
            ryd.start()
            rzd.start()
            yd.append(ryd)
            zd.append(rzd)

        yf, zf = [], []
        for c in range(C // 2):
            zd[c].wait_recv()
            r = pltpu.make_async_remote_copy(
                src_ref=out_ref.at[rows(b_z, c), :],
                dst_ref=out_ref.at[rows(b_z, c), :],
                send_sem=yf_s.at[c],
                recv_sem=yf_r.at[c],
                device_id=yn,
                device_id_type=_DeviceIdType.MESH,
            )
            r.start()
            yf.append(r)
        for c in range(C // 2, C):
            yd[c].wait_recv()
            r = pltpu.make_async_remote_copy(
                src_ref=out_ref.at[rows(b_y, c), :],
                dst_ref=out_ref.at[rows(b_y, c), :],
                send_sem=zf_s.at[c - C // 2],
                recv_sem=zf_r.at[c - C // 2],
                device_id=zn,
                device_id_type=_DeviceIdType.MESH,
            )
            r.start()
            zf.append(r)

        for c in range(C // 2):
            yd[c].wait_recv()
        for c in range(C // 2, C):
            zd[c].wait_recv()
        for r in yf + zf:
            r.wait_recv()
        for r in x_rdma + yd + zd + yf + zf:
            r.wait_send()

    return pl.pallas_call(
        body,
        out_shape=jax.ShapeDtypeStruct((M, D), jnp.float32),
        in_specs=[
            pl.BlockSpec(memory_space=pltpu.VMEM),
            pl.BlockSpec(memory_space=pltpu.VMEM),
            pl.BlockSpec(memory_space=pltpu.VMEM),
        ],
        out_specs=pl.BlockSpec(memory_space=pltpu.VMEM),
        scratch_shapes=[
            pltpu.VMEM((BM, D), jnp.float32),
            pltpu.SemaphoreType.DMA((C,)),
            pltpu.SemaphoreType.DMA((C,)),
            pltpu.SemaphoreType.DMA((C,)),
            pltpu.SemaphoreType.DMA((C,)),
            pltpu.SemaphoreType.DMA((C,)),
            pltpu.SemaphoreType.DMA((C,)),
            pltpu.SemaphoreType.DMA((C // 2,)),
            pltpu.SemaphoreType.DMA((C // 2,)),
            pltpu.SemaphoreType.DMA((C // 2,)),
            pltpu.SemaphoreType.DMA((C // 2,)),
        ],
        compiler_params=_CompilerParams(collective_id=0),
    )(p, resid, g)


# device time: 33369 ns/iter; 1.7053x vs baseline; 1.0341x over previous
import jax
import jax.numpy as jnp
from jax import lax
from jax.experimental import pallas as pl
from jax.experimental.pallas import tpu as pltpu

_sem_signal = getattr(pl, "semaphore_signal", None) or pltpu.semaphore_signal
_sem_wait = getattr(pl, "semaphore_wait", None) or pltpu.semaphore_wait
_DeviceIdType = getattr(pl, "DeviceIdType", None) or pltpu.DeviceIdType
_CompilerParams = getattr(pltpu, "CompilerParams", None) or getattr(
    pltpu, "TPUCompilerParams"
)

M = 1024
D = 1024
EPS = 1e-6
NB = 4
BM = M // NB
C = 8
CM = BM // C


def kernel(partial, resid, gamma):
    p = partial.reshape(M, D)
    g = gamma.reshape(1, D)

    def body(
        p_ref, r_ref, g_ref, out_ref, comm_x,
        sx_s, sx_r, yd_s, yd_r, zd_s, zd_r, yf_s, yf_r, zf_s, zf_r,
    ):
        my_x = lax.axis_index("x")
        my_y = lax.axis_index("y")
        my_z = lax.axis_index("z")
        xn = (1 - my_x, my_y, my_z)
        yn = (my_x, 1 - my_y, my_z)
        zn = (my_x, my_y, 1 - my_z)

        b = 2 * my_y + my_z
        b_y = 2 * (1 - my_y) + my_z
        b_z = 2 * my_y + (1 - my_z)

        def rows(blk, c):
            return pl.ds(blk * BM + c * CM, CM)

        barrier_sem = pltpu.get_barrier_semaphore()
        for nbr in (xn, yn, zn):
            _sem_signal(
                barrier_sem, inc=1, device_id=nbr,
                device_id_type=_DeviceIdType.MESH,
            )
        _sem_wait(barrier_sem, 3)

        x_rdma = []
        for c in range(C):
            r = pltpu.make_async_remote_copy(
                src_ref=p_ref.at[rows(b, c), :],
                dst_ref=comm_x.at[pl.ds(c * CM, CM), :],
                send_sem=sx_s.at[c],
                recv_sem=sx_r.at[c],
                device_id=xn,
                device_id_type=_DeviceIdType.MESH,
            )
            r.start()
            x_rdma.append(r)

        def direct(blk, c, sems_s, sems_r, nbr):
            return pltpu.make_async_remote_copy(
                src_ref=out_ref.at[rows(blk, c), :],
                dst_ref=out_ref.at[rows(blk, c), :],
                send_sem=sems_s.at[c],
                recv_sem=sems_r.at[c],
                device_id=nbr,
                device_id_type=_DeviceIdType.MESH,
            )

        yd, zd = [], []
        for c in range(C):
            x_rdma[c].wait_recv()
            y = p_ref[rows(b, c), :] + comm_x[pl.ds(c * CM, CM), :] \
                + r_ref[rows(b, c), :]
            ms = jnp.mean(y * y, axis=-1, keepdims=True)
            out_ref[rows(b, c), :] = y * lax.rsqrt(ms + EPS) * g_ref[...]
            ryd = direct(b, c, yd_s, yd_r, yn)
            rzd = direct(b, c, zd_s, zd_r, zn)
            ryd.start()
            rzd.start()
            yd.append(ryd)
            zd.append(rzd)

        yf, zf = [], []
        for c in range(C // 2):
            zd[c].wait_recv()
            r = pltpu.make_async_remote_copy(
                src_ref=out_ref.at[rows(b_z, c), :],
                dst_ref=out_ref.at[rows(b_z, c), :],
                send_sem=yf_s.at[c],
                recv_sem=yf_r.at[c],
                device_id=yn,
                device_id_type=_DeviceIdType.MESH,
            )
            r.start()
            yf.append(r)
        for c in range(C // 2, C):
            yd[c].wait_recv()
            r = pltpu.make_async_remote_copy(
                src_ref=out_ref.at[rows(b_y, c), :],
                dst_ref=out_ref.at[rows(b_y, c), :],
                send_sem=zf_s.at[c - C // 2],
                recv_sem=zf_r.at[c - C // 2],
                device_id=zn,
                device_id_type=_DeviceIdType.MESH,
            )
            r.start()
            zf.append(r)

        for c in range(C // 2):
            yd[c].wait_recv()
        for c in range(C // 2, C):
            zd[c].wait_recv()
        for r in yf + zf:
            r.wait_recv()
        for r in x_rdma + yd + zd + yf + zf:
            r.wait_send()

    return pl.pallas_call(
        body,
        out_shape=jax.ShapeDtypeStruct((M, D), jnp.float32),
        in_specs=[
            pl.BlockSpec(memory_space=pltpu.VMEM),
            pl.BlockSpec(memory_space=pltpu.VMEM),
            pl.BlockSpec(memory_space=pltpu.VMEM),
        ],
        out_specs=pl.BlockSpec(memory_space=pltpu.VMEM),
        scratch_shapes=[
            pltpu.VMEM((BM, D), jnp.float32),
            pltpu.SemaphoreType.DMA((C,)),
            pltpu.SemaphoreType.DMA((C,)),
            pltpu.SemaphoreType.DMA((C,)),
            pltpu.SemaphoreType.DMA((C,)),
            pltpu.SemaphoreType.DMA((C,)),
            pltpu.SemaphoreType.DMA((C,)),
            pltpu.SemaphoreType.DMA((C // 2,)),
            pltpu.SemaphoreType.DMA((C // 2,)),
            pltpu.SemaphoreType.DMA((C // 2,)),
            pltpu.SemaphoreType.DMA((C // 2,)),
        ],
        compiler_params=_CompilerParams(collective_id=0),
    )(p, resid, g)
